# baseline (speedup 1.0000x reference)
import functools

import jax
import jax.numpy as jnp
from jax import lax
from jax.experimental import pallas as pl
from jax.experimental.pallas import tpu as pltpu
from jax.experimental.pallas import tpu_sc as plsc

F = 26
V = 100000
E = 16
ND = 11
B = 16384
H1, H2, OUT = 128, 64, 10

NC, NS = 2, 16
HB = B // NC
VA = 49920
VB = V - VA


def _sc_body(tt, codes, bi_out, row_a, row_b, codes_v, acc_s, acc_q,
             sem_a, sem_b):
    e = lax.axis_index("s")
    ch = lax.axis_index("c")
    zeros16 = jnp.zeros((16,), jnp.int32)

    def src_a(f):
        return tt.at[f, pl.ds(e, 1), pl.ds(0, VA)]

    def src_b(f):
        return tt.at[f, pl.ds(e, 1), pl.ds(VA, VB)]

    def compute(row, lo, size, first):
        def grp(g, cr):
            sl = pl.ds(g * 16, 16)
            cd = codes_v[sl]
            rel = cd - lo
            cl = jnp.minimum(jnp.maximum(rel, 0), size - 1)
            v = plsc.load_gather(row, [zeros16, cl])
            m = (rel >= 0) & (rel < size)
            v = jnp.where(m, v, 0.0)
            if first:
                acc_s[0, sl] = v
                acc_q[sl] = v * v
            else:
                acc_s[0, sl] += v
                acc_q[sl] += v * v
            return cr
        lax.fori_loop(0, HB // 16, grp, 0)

    pltpu.async_copy(src_a(0), row_a, sem_a)
    pltpu.async_copy(src_b(0), row_b, sem_b)

    for f in range(F):
        pltpu.sync_copy(codes.at[f, pl.ds(ch * HB, HB)], codes_v)

        pltpu.make_async_copy(src_a(f), row_a, sem_a).wait()
        compute(row_a, 0, VA, first=(f == 0))
        if f + 1 < F:
            pltpu.async_copy(src_a(f + 1), row_a, sem_a)

        pltpu.make_async_copy(src_b(f), row_b, sem_b).wait()
        compute(row_b, VA, VB, first=False)
        if f + 1 < F:
            pltpu.async_copy(src_b(f + 1), row_b, sem_b)

    def fin(g, cr):
        sl = pl.ds(g * 16, 16)
        s = acc_s[0, sl]
        q = acc_q[sl]
        acc_s[0, sl] = 0.5 * (s * s - q)
        return cr
    lax.fori_loop(0, HB // 16, fin, 0)

    pltpu.sync_copy(acc_s, bi_out.at[pl.ds(e, 1), pl.ds(ch * HB, HB)])


_sc_pool = functools.partial(
    pl.kernel,
    out_type=jax.ShapeDtypeStruct((E, B), jnp.float32),
    mesh=plsc.VectorSubcoreMesh(core_axis_name="c", subcore_axis_name="s"),
    scratch_types=[
        pltpu.VMEM((1, VA), jnp.float32),
        pltpu.VMEM((1, VB), jnp.float32),
        pltpu.VMEM((HB,), jnp.int32),
        pltpu.VMEM((1, HB), jnp.float32),
        pltpu.VMEM((HB,), jnp.float32),
        pltpu.SemaphoreType.DMA,
        pltpu.SemaphoreType.DMA,
    ],
    compiler_params=pltpu.CompilerParams(needs_layout_passes=False),
)(_sc_body)


BM = 2048


def _mlp_body(dense_ref, bit_ref, w1a_ref, w1b_ref, b1_ref, w2_ref, b2_ref,
              w3_ref, b3_ref, out_ref):
    h = jnp.dot(dense_ref[...], w1a_ref[...], preferred_element_type=jnp.float32)
    h += lax.dot_general(bit_ref[...], w1b_ref[...],
                         (((0,), (0,)), ((), ())),
                         preferred_element_type=jnp.float32)
    h = jnp.maximum(h + b1_ref[...], 0.0)
    h = jnp.dot(h, w2_ref[...], preferred_element_type=jnp.float32)
    h = jnp.maximum(h + b2_ref[...], 0.0)
    out_ref[...] = (
        jnp.dot(h, w3_ref[...], preferred_element_type=jnp.float32)
        + b3_ref[...])


def _mlp(dense, bi_t, W1a, W1b, b1, W2, b2, W3, b3):
    grid = (B // BM,)
    full = lambda shape: pl.BlockSpec(shape, lambda i: (0, 0))
    return pl.pallas_call(
        _mlp_body,
        grid=grid,
        in_specs=[
            pl.BlockSpec((BM, ND), lambda i: (i, 0)),
            pl.BlockSpec((E, BM), lambda i: (0, i)),
            full((ND, H1)),
            full((E, H1)),
            full((1, H1)),
            full((H1, H2)),
            full((1, H2)),
            full((H2, OUT)),
            full((1, OUT)),
        ],
        out_specs=pl.BlockSpec((BM, OUT), lambda i: (i, 0)),
        out_shape=jax.ShapeDtypeStruct((B, OUT), jnp.float32),
    )(dense, bi_t, W1a, W1b, b1, W2, b2, W3, b3)


def kernel(target_x, tables, W1, b1, W2, b2, W3, b3):
    dense = target_x[:, :ND]
    sparse = target_x[:, ND:].astype(jnp.int32)
    codes_t = jnp.transpose(sparse, (1, 0))
    tt = jnp.transpose(tables, (0, 2, 1))

    bi_t = _sc_pool(tt, codes_t)

    return _mlp(dense, bi_t, W1[:ND], W1[ND:], b1[None, :], W2, b2[None, :],
                W3, b3[None, :])


# speedup vs baseline: 1.1031x; 1.1031x over previous
import functools

import jax
import jax.numpy as jnp
from jax import lax
from jax.experimental import pallas as pl
from jax.experimental.pallas import tpu as pltpu
from jax.experimental.pallas import tpu_sc as plsc

F = 26
V = 100000
E = 16
ND = 11
B = 16384
H1, H2, OUT = 128, 64, 10

NC, NS = 2, 16
HB = B // NC


def _sc_body(tt, codes, bi_out, row_v, codes_v, acc_s, acc_q, sem):
    e = lax.axis_index("s")
    ch = lax.axis_index("c")
    zeros16 = jnp.zeros((16,), jnp.int32)

    def src(f):
        return tt.at[f, pl.ds(e, 1), :]

    def compute(first):
        def grp(g, cr):
            sl = pl.ds(g * 16, 16)
            v = plsc.load_gather(row_v, [zeros16, codes_v[sl]])
            if first:
                acc_s[0, sl] = v
                acc_q[sl] = v * v
            else:
                acc_s[0, sl] += v
                acc_q[sl] += v * v
            return cr
        lax.fori_loop(0, HB // 16, grp, 0)

    pltpu.async_copy(src(0), row_v, sem)
    pltpu.sync_copy(codes.at[0, pl.ds(ch * HB, HB)], codes_v)

    for f in range(F):
        pltpu.make_async_copy(src(f), row_v, sem).wait()
        compute(first=(f == 0))
        if f + 1 < F:
            pltpu.async_copy(src(f + 1), row_v, sem)
            pltpu.sync_copy(codes.at[f + 1, pl.ds(ch * HB, HB)], codes_v)

    def fin(g, cr):
        sl = pl.ds(g * 16, 16)
        s = acc_s[0, sl]
        q = acc_q[sl]
        acc_s[0, sl] = 0.5 * (s * s - q)
        return cr
    lax.fori_loop(0, HB // 16, fin, 0)

    pltpu.sync_copy(acc_s, bi_out.at[pl.ds(e, 1), pl.ds(ch * HB, HB)])


_sc_pool = functools.partial(
    pl.kernel,
    out_type=jax.ShapeDtypeStruct((E, B), jnp.float32),
    mesh=plsc.VectorSubcoreMesh(core_axis_name="c", subcore_axis_name="s"),
    scratch_types=[
        pltpu.VMEM((1, V), jnp.float32),
        pltpu.VMEM((HB,), jnp.int32),
        pltpu.VMEM((1, HB), jnp.float32),
        pltpu.VMEM((HB,), jnp.float32),
        pltpu.SemaphoreType.DMA,
    ],
    compiler_params=pltpu.CompilerParams(needs_layout_passes=False),
)(_sc_body)


BM = 2048


def _mlp_body(dense_ref, bit_ref, w1a_ref, w1b_ref, b1_ref, w2_ref, b2_ref,
              w3_ref, b3_ref, out_ref):
    h = jnp.dot(dense_ref[...], w1a_ref[...], preferred_element_type=jnp.float32)
    h += lax.dot_general(bit_ref[...], w1b_ref[...],
                         (((0,), (0,)), ((), ())),
                         preferred_element_type=jnp.float32)
    h = jnp.maximum(h + b1_ref[...], 0.0)
    h = jnp.dot(h, w2_ref[...], preferred_element_type=jnp.float32)
    h = jnp.maximum(h + b2_ref[...], 0.0)
    out_ref[...] = (
        jnp.dot(h, w3_ref[...], preferred_element_type=jnp.float32)
        + b3_ref[...])


def _mlp(dense, bi_t, W1a, W1b, b1, W2, b2, W3, b3):
    grid = (B // BM,)
    full = lambda shape: pl.BlockSpec(shape, lambda i: (0, 0))
    return pl.pallas_call(
        _mlp_body,
        grid=grid,
        in_specs=[
            pl.BlockSpec((BM, ND), lambda i: (i, 0)),
            pl.BlockSpec((E, BM), lambda i: (0, i)),
            full((ND, H1)),
            full((E, H1)),
            full((1, H1)),
            full((H1, H2)),
            full((1, H2)),
            full((H2, OUT)),
            full((1, OUT)),
        ],
        out_specs=pl.BlockSpec((BM, OUT), lambda i: (i, 0)),
        out_shape=jax.ShapeDtypeStruct((B, OUT), jnp.float32),
    )(dense, bi_t, W1a, W1b, b1, W2, b2, W3, b3)


def kernel(target_x, tables, W1, b1, W2, b2, W3, b3):
    dense = target_x[:, :ND]
    sparse = target_x[:, ND:].astype(jnp.int32)
    codes_t = jnp.transpose(sparse, (1, 0))
    tt = jnp.transpose(tables, (0, 2, 1))

    bi_t = _sc_pool(tt, codes_t)

    return _mlp(dense, bi_t, W1[:ND], W1[ND:], b1[None, :], W2, b2[None, :],
                W3, b3[None, :])
